# initial kernel scaffold (unmeasured)
import jax
import jax.numpy as jnp
from jax import lax
from jax.experimental import pallas as pl
from jax.experimental.pallas import tpu as pltpu

M = 2048
D = 2048


def kernel(partial, resid, gamma):
    gamma2d = gamma.reshape(1, D)

    def body(p_ref, r_ref, g_ref, o_ref, comm_ref, send_sem, recv_sem):
        my_x = lax.axis_index("x")
        my_y = lax.axis_index("y")
        my_z = lax.axis_index("z")
        nbr = (my_x, my_y, 1 - my_z)

        barrier_sem = pltpu.get_barrier_semaphore()
        pl.semaphore_signal(
            barrier_sem, inc=1, device_id=nbr,
            device_id_type=pl.DeviceIdType.MESH,
        )
        pl.semaphore_wait(barrier_sem, 1)

        comm_ref[0, :, :] = p_ref[0, :, :].astype(jnp.bfloat16)
        rdma = pltpu.make_async_remote_copy(
            src_ref=comm_ref.at[0],
            dst_ref=comm_ref.at[1],
            send_sem=send_sem,
            recv_sem=recv_sem,
            device_id=nbr,
            device_id_type=pl.DeviceIdType.MESH,
        )
        rdma.start()
        rdma.wait()

        y = (
            p_ref[0, :, :]
            + comm_ref[1, :, :].astype(jnp.float32)
            + r_ref[:, :]
        )
        rms = jnp.sqrt(jnp.mean(y * y, axis=-1, keepdims=True) + 1e-6)
        o_ref[:, :] = y / rms * g_ref[0, :][None, :]

    return pl.pallas_call(
        body,
        out_shape=jax.ShapeDtypeStruct((M, D), jnp.float32),
        in_specs=[
            pl.BlockSpec(memory_space=pltpu.VMEM),
            pl.BlockSpec(memory_space=pltpu.VMEM),
            pl.BlockSpec(memory_space=pltpu.VMEM),
        ],
        out_specs=pl.BlockSpec(memory_space=pltpu.VMEM),
        scratch_shapes=[
            pltpu.VMEM((2, M, D), jnp.bfloat16),
            pltpu.SemaphoreType.DMA,
            pltpu.SemaphoreType.DMA,
        ],
        compiler_params=pltpu.CompilerParams(collective_id=0),
    )(partial, resid, gamma2d)


# baseline (device time: 84174 ns/iter reference)
import jax
import jax.numpy as jnp
from jax import lax
from jax.experimental import pallas as pl
from jax.experimental.pallas import tpu as pltpu

M = 2048
D = 2048
MQ = M // 4


def kernel(partial, resid, gamma):
    gamma2d = gamma.reshape(1, D)

    def body(p_hbm, r_hbm, g_ref, o_ref,
             pq_ref, rq_ref, zsend_ref, zrecv_ref,
             copy_sems, send_sems, recv_sems):
        my_x = lax.axis_index("x")
        my_y = lax.axis_index("y")
        my_z = lax.axis_index("z")
        q = 2 * my_x + my_y
        row0 = q * MQ

        z_nbr = (my_x, my_y, 1 - my_z)
        x_nbr = (1 - my_x, my_y, my_z)
        y_nbr = (my_x, 1 - my_y, my_z)
        d_nbr = (1 - my_x, 1 - my_y, my_z)
        peers = [z_nbr, x_nbr, y_nbr, d_nbr]

        cp_p = pltpu.make_async_copy(
            p_hbm.at[0, pl.ds(row0, MQ), :], pq_ref, copy_sems.at[0])
        cp_r = pltpu.make_async_copy(
            r_hbm.at[pl.ds(row0, MQ), :], rq_ref, copy_sems.at[1])
        cp_p.start()
        cp_r.start()
        cp_p.wait()
        cp_r.wait()

        barrier_sem = pltpu.get_barrier_semaphore()
        for nbr in peers:
            pl.semaphore_signal(
                barrier_sem, inc=1, device_id=nbr,
                device_id_type=pl.DeviceIdType.MESH,
            )
        pl.semaphore_wait(barrier_sem, len(peers))

        zsend_ref[:, :] = pq_ref[:, :].astype(jnp.bfloat16)
        z_rdma = pltpu.make_async_remote_copy(
            src_ref=zsend_ref,
            dst_ref=zrecv_ref,
            send_sem=send_sems.at[0],
            recv_sem=recv_sems.at[0],
            device_id=z_nbr,
            device_id_type=pl.DeviceIdType.MESH,
        )
        z_rdma.start()
        z_rdma.wait()

        y = pq_ref[:, :] + zrecv_ref[:, :].astype(jnp.float32) + rq_ref[:, :]
        rms = jnp.sqrt(jnp.mean(y * y, axis=-1, keepdims=True) + 1e-6)
        o_ref[pl.ds(row0, MQ), :] = (y / rms * g_ref[0, :][None, :]).astype(
            jnp.bfloat16)

        plane = [(x_nbr, 1), (y_nbr, 2), (d_nbr, 3)]
        rdmas = []
        for nbr, s in plane:
            rdma = pltpu.make_async_remote_copy(
                src_ref=o_ref.at[pl.ds(row0, MQ), :],
                dst_ref=o_ref.at[pl.ds(row0, MQ), :],
                send_sem=send_sems.at[s],
                recv_sem=recv_sems.at[s],
                device_id=nbr,
                device_id_type=pl.DeviceIdType.MESH,
            )
            rdma.start()
            rdmas.append(rdma)
        for rdma in rdmas:
            rdma.wait_send()
        for nbr, s in plane:
            nq = 2 * nbr[0] + nbr[1]
            recv = pltpu.make_async_remote_copy(
                src_ref=o_ref.at[pl.ds(row0, MQ), :],
                dst_ref=o_ref.at[pl.ds(nq * MQ, MQ), :],
                send_sem=send_sems.at[s],
                recv_sem=recv_sems.at[s],
                device_id=nbr,
                device_id_type=pl.DeviceIdType.MESH,
            )
            recv.wait_recv()

    return pl.pallas_call(
        body,
        out_shape=jax.ShapeDtypeStruct((M, D), jnp.bfloat16),
        in_specs=[
            pl.BlockSpec(memory_space=pl.ANY),
            pl.BlockSpec(memory_space=pl.ANY),
            pl.BlockSpec(memory_space=pltpu.VMEM),
        ],
        out_specs=pl.BlockSpec(memory_space=pltpu.VMEM),
        scratch_shapes=[
            pltpu.VMEM((MQ, D), jnp.float32),
            pltpu.VMEM((MQ, D), jnp.float32),
            pltpu.VMEM((MQ, D), jnp.bfloat16),
            pltpu.VMEM((MQ, D), jnp.bfloat16),
            pltpu.SemaphoreType.DMA((2,)),
            pltpu.SemaphoreType.DMA((4,)),
            pltpu.SemaphoreType.DMA((4,)),
        ],
        compiler_params=pltpu.CompilerParams(collective_id=0),
    )(partial, resid, gamma2d)


# device time: 56579 ns/iter; 1.4877x vs baseline; 1.4877x over previous
import jax
import jax.numpy as jnp
from jax import lax
from jax.experimental import pallas as pl
from jax.experimental.pallas import tpu as pltpu

M = 2048
D = 2048
MQ = M // 4
NC = 4
R = MQ // NC
H = R // 2

CH_Z, CH_X, CH_Y, CH_D, CH_F = range(5)


def kernel(partial, resid, gamma):
    gamma2d = gamma.reshape(1, D)

    def body(p_hbm, r_hbm, g_ref, o_ref,
             pq_ref, rq_ref, zsend_ref, zrecv_ref,
             copy_sems, send_sems, recv_sems):
        my_x = lax.axis_index("x")
        my_y = lax.axis_index("y")
        my_z = lax.axis_index("z")
        q = 2 * my_x + my_y
        row0 = q * MQ

        z_nbr = (my_x, my_y, 1 - my_z)
        x_nbr = (1 - my_x, my_y, my_z)
        y_nbr = (my_x, 1 - my_y, my_z)
        d_nbr = (1 - my_x, 1 - my_y, my_z)

        qx = q ^ 2
        qy = q ^ 1
        qd = q ^ 3

        def rdma(src, dst, ch, c, dev):
            return pltpu.make_async_remote_copy(
                src_ref=src, dst_ref=dst,
                send_sem=send_sems.at[ch, c], recv_sem=recv_sems.at[ch, c],
                device_id=dev, device_id_type=pl.DeviceIdType.MESH,
            )

        p_cps, r_cps = [], []
        for c in range(NC):
            cp = pltpu.make_async_copy(
                p_hbm.at[0, pl.ds(row0 + c * R, R), :],
                pq_ref.at[pl.ds(c * R, R), :], copy_sems.at[0, c])
            cp.start()
            p_cps.append(cp)
            cp = pltpu.make_async_copy(
                r_hbm.at[pl.ds(row0 + c * R, R), :],
                rq_ref.at[pl.ds(c * R, R), :], copy_sems.at[1, c])
            cp.start()
            r_cps.append(cp)

        barrier_sem = pltpu.get_barrier_semaphore()
        for nbr in [z_nbr, x_nbr, y_nbr, d_nbr]:
            pl.semaphore_signal(
                barrier_sem, inc=1, device_id=nbr,
                device_id_type=pl.DeviceIdType.MESH,
            )
        pl.semaphore_wait(barrier_sem, 4)

        z_rdmas = []
        for c in range(NC):
            p_cps[c].wait()
            zsend_ref[c, :, :] = pq_ref[
                pl.ds(c * R, R), :].astype(jnp.bfloat16)
            zr = rdma(zsend_ref.at[c], zrecv_ref.at[c], CH_Z, c, z_nbr)
            zr.start()
            z_rdmas.append(zr)

        sends = []
        for c in range(NC):
            z_rdmas[c].wait_recv()
            r_cps[c].wait()
            y = (pq_ref[pl.ds(c * R, R), :]
                 + zrecv_ref[c, :, :].astype(jnp.float32)
                 + rq_ref[pl.ds(c * R, R), :])
            rms = jnp.sqrt(jnp.mean(y * y, axis=-1, keepdims=True) + 1e-6)
            o_ref[pl.ds(row0 + c * R, R), :] = (
                y / rms * g_ref[0, :][None, :]).astype(jnp.bfloat16)

            chunk = o_ref.at[pl.ds(row0 + c * R, R), :]
            for ch, nbr in ((CH_X, x_nbr), (CH_Y, y_nbr)):
                s = rdma(chunk, chunk, ch, c, nbr)
                s.start()
                sends.append(s)
            half = o_ref.at[pl.ds(row0 + c * R + my_z * H, H), :]
            s = rdma(half, half, CH_D, c, d_nbr)
            s.start()
            sends.append(s)

        for c in range(NC):
            d_half = o_ref.at[pl.ds(qd * MQ + c * R + my_z * H, H), :]
            rdma(d_half, d_half, CH_D, c, d_nbr).wait_recv()
            s = rdma(d_half, d_half, CH_F, c, z_nbr)
            s.start()
            sends.append(s)

        for c in range(NC):
            rdma(o_ref.at[pl.ds(qx * MQ + c * R, R), :],
                 o_ref.at[pl.ds(qx * MQ + c * R, R), :],
                 CH_X, c, x_nbr).wait_recv()
            rdma(o_ref.at[pl.ds(qy * MQ + c * R, R), :],
                 o_ref.at[pl.ds(qy * MQ + c * R, R), :],
                 CH_Y, c, y_nbr).wait_recv()
            fwd_rows = qd * MQ + c * R + (1 - my_z) * H
            rdma(o_ref.at[pl.ds(fwd_rows, H), :],
                 o_ref.at[pl.ds(fwd_rows, H), :],
                 CH_F, c, z_nbr).wait_recv()
        for s in z_rdmas:
            s.wait_send()
        for s in sends:
            s.wait_send()

    return pl.pallas_call(
        body,
        out_shape=jax.ShapeDtypeStruct((M, D), jnp.bfloat16),
        in_specs=[
            pl.BlockSpec(memory_space=pl.ANY),
            pl.BlockSpec(memory_space=pl.ANY),
            pl.BlockSpec(memory_space=pltpu.VMEM),
        ],
        out_specs=pl.BlockSpec(memory_space=pltpu.VMEM),
        scratch_shapes=[
            pltpu.VMEM((MQ, D), jnp.float32),
            pltpu.VMEM((MQ, D), jnp.float32),
            pltpu.VMEM((NC, R, D), jnp.bfloat16),
            pltpu.VMEM((NC, R, D), jnp.bfloat16),
            pltpu.SemaphoreType.DMA((2, NC)),
            pltpu.SemaphoreType.DMA((5, NC)),
            pltpu.SemaphoreType.DMA((5, NC)),
        ],
        compiler_params=pltpu.CompilerParams(collective_id=0),
    )(partial, resid, gamma2d)


# device time: 52509 ns/iter; 1.6030x vs baseline; 1.0775x over previous
import jax
import jax.numpy as jnp
from jax import lax
from jax.experimental import pallas as pl
from jax.experimental.pallas import tpu as pltpu

M = 2048
D = 2048
MQ = M // 4
NC = 8
R = MQ // NC
H = R // 2

CH_Z, CH_X, CH_Y, CH_D, CH_F = range(5)


def kernel(partial, resid, gamma):
    gamma2d = gamma.reshape(1, D)

    def body(p_hbm, r_hbm, g_ref, o_ref,
             pq_ref, rq_ref, zsend_ref, zrecv_ref,
             copy_sems, send_sems, recv_sems):
        my_x = lax.axis_index("x")
        my_y = lax.axis_index("y")
        my_z = lax.axis_index("z")
        q = 2 * my_x + my_y
        row0 = q * MQ

        z_nbr = (my_x, my_y, 1 - my_z)
        x_nbr = (1 - my_x, my_y, my_z)
        y_nbr = (my_x, 1 - my_y, my_z)
        d_nbr = (1 - my_x, 1 - my_y, my_z)

        qx = q ^ 2
        qy = q ^ 1
        qd = q ^ 3

        def rdma(src, dst, ch, c, dev):
            return pltpu.make_async_remote_copy(
                src_ref=src, dst_ref=dst,
                send_sem=send_sems.at[ch, c], recv_sem=recv_sems.at[ch, c],
                device_id=dev, device_id_type=pl.DeviceIdType.MESH,
            )

        p_cps, r_cps = [], []
        for c in range(NC):
            cp = pltpu.make_async_copy(
                p_hbm.at[0, pl.ds(row0 + c * R, R), :],
                pq_ref.at[pl.ds(c * R, R), :], copy_sems.at[0, c])
            cp.start()
            p_cps.append(cp)
            cp = pltpu.make_async_copy(
                r_hbm.at[pl.ds(row0 + c * R, R), :],
                rq_ref.at[pl.ds(c * R, R), :], copy_sems.at[1, c])
            cp.start()
            r_cps.append(cp)

        barrier_sem = pltpu.get_barrier_semaphore()
        for nbr in [z_nbr, x_nbr, y_nbr, d_nbr]:
            pl.semaphore_signal(
                barrier_sem, inc=1, device_id=nbr,
                device_id_type=pl.DeviceIdType.MESH,
            )
        pl.semaphore_wait(barrier_sem, 4)

        z_rdmas = []
        for c in range(NC):
            p_cps[c].wait()
            zsend_ref[c, :, :] = pq_ref[
                pl.ds(c * R, R), :].astype(jnp.bfloat16)
            zr = rdma(zsend_ref.at[c], zrecv_ref.at[c], CH_Z, c, z_nbr)
            zr.start()
            z_rdmas.append(zr)

        sends = []
        for c in range(NC):
            z_rdmas[c].wait_recv()
            r_cps[c].wait()
            y = (pq_ref[pl.ds(c * R, R), :]
                 + zrecv_ref[c, :, :].astype(jnp.float32)
                 + rq_ref[pl.ds(c * R, R), :])
            rms = jnp.sqrt(jnp.mean(y * y, axis=-1, keepdims=True) + 1e-6)
            o_ref[pl.ds(row0 + c * R, R), :] = (
                y / rms * g_ref[0, :][None, :]).astype(jnp.bfloat16)

            chunk = o_ref.at[pl.ds(row0 + c * R, R), :]
            for ch, nbr in ((CH_X, x_nbr), (CH_Y, y_nbr)):
                s = rdma(chunk, chunk, ch, c, nbr)
                s.start()
                sends.append(s)
            half = o_ref.at[pl.ds(row0 + c * R + my_z * H, H), :]
            s = rdma(half, half, CH_D, c, d_nbr)
            s.start()
            sends.append(s)

        for c in range(NC):
            d_half = o_ref.at[pl.ds(qd * MQ + c * R + my_z * H, H), :]
            rdma(d_half, d_half, CH_D, c, d_nbr).wait_recv()
            s = rdma(d_half, d_half, CH_F, c, z_nbr)
            s.start()
            sends.append(s)

        for c in range(NC):
            rdma(o_ref.at[pl.ds(qx * MQ + c * R, R), :],
                 o_ref.at[pl.ds(qx * MQ + c * R, R), :],
                 CH_X, c, x_nbr).wait_recv()
            rdma(o_ref.at[pl.ds(qy * MQ + c * R, R), :],
                 o_ref.at[pl.ds(qy * MQ + c * R, R), :],
                 CH_Y, c, y_nbr).wait_recv()
            fwd_rows = qd * MQ + c * R + (1 - my_z) * H
            rdma(o_ref.at[pl.ds(fwd_rows, H), :],
                 o_ref.at[pl.ds(fwd_rows, H), :],
                 CH_F, c, z_nbr).wait_recv()
        for s in z_rdmas:
            s.wait_send()
        for s in sends:
            s.wait_send()

    return pl.pallas_call(
        body,
        out_shape=jax.ShapeDtypeStruct((M, D), jnp.bfloat16),
        in_specs=[
            pl.BlockSpec(memory_space=pl.ANY),
            pl.BlockSpec(memory_space=pl.ANY),
            pl.BlockSpec(memory_space=pltpu.VMEM),
        ],
        out_specs=pl.BlockSpec(memory_space=pltpu.VMEM),
        scratch_shapes=[
            pltpu.VMEM((MQ, D), jnp.float32),
            pltpu.VMEM((MQ, D), jnp.float32),
            pltpu.VMEM((NC, R, D), jnp.bfloat16),
            pltpu.VMEM((NC, R, D), jnp.bfloat16),
            pltpu.SemaphoreType.DMA((2, NC)),
            pltpu.SemaphoreType.DMA((5, NC)),
            pltpu.SemaphoreType.DMA((5, NC)),
        ],
        compiler_params=pltpu.CompilerParams(collective_id=0),
    )(partial, resid, gamma2d)
